# baseline (speedup 1.0000x reference)
import jax
import jax.numpy as jnp
from jax import lax
from jax.experimental import pallas as pl
from jax.experimental.pallas import tpu as pltpu
from jax.experimental.pallas import tpu_sc as plsc

N = 10000
S = 32
D = 128
DOUT = 128
LANES = 16
NJ = D // LANES
NC = 2
NS = 16
NW = NC * NS

NSC = 1600
NTC = N - NSC
CH = 8
CHUNK_ROWS = CH * S
NCH = NSC // CH
TOT = 2 * NCH
NB = 2
SLOTS = NB * ((TOT + NB * NW - 1) // (NB * NW))


def _sc_agg_body(src_neg_hbm, dst_neg_hbm, src_sum_hbm, dst_sum_hbm,
                 *scratch):
    wid = lax.axis_index("s") * NC + lax.axis_index("c")
    bufs = scratch[0:NB]
    obufs = scratch[NB:2 * NB]
    sins = scratch[2 * NB:3 * NB]
    souts = scratch[3 * NB:4 * NB]

    def item_of(j):
        return jnp.minimum(wid + j * NW, TOT - 1)

    def start_in(j, b):
        k = item_of(j)

        @pl.when(k < NCH)
        def _():
            pltpu.async_copy(
                src_neg_hbm.at[pl.ds(k * CHUNK_ROWS, CHUNK_ROWS)],
                bufs[b], sins[b])

        @pl.when(k >= NCH)
        def _():
            pltpu.async_copy(
                dst_neg_hbm.at[pl.ds((k - NCH) * CHUNK_ROWS, CHUNK_ROWS)],
                bufs[b], sins[b])

    def wait_in(b):
        pltpu.make_async_copy(
            src_neg_hbm.at[pl.ds(0, CHUNK_ROWS)], bufs[b], sins[b]).wait()

    def start_out(j, b):
        k = item_of(j)

        @pl.when(k < NCH)
        def _():
            pltpu.async_copy(obufs[b], src_sum_hbm.at[pl.ds(k * CH, CH)],
                             souts[b])

        @pl.when(k >= NCH)
        def _():
            pltpu.async_copy(obufs[b],
                             dst_sum_hbm.at[pl.ds((k - NCH) * CH, CH)],
                             souts[b])

    def wait_out(b):
        pltpu.make_async_copy(
            obufs[b], src_sum_hbm.at[pl.ds(0, CH)], souts[b]).wait()

    def compute(b):
        buf, obuf = bufs[b], obufs[b]

        def node_body(n, _):
            base = n * S

            def row_body(s, accs):
                return tuple(
                    accs[j] + buf[base + s, pl.ds(j * LANES, LANES)]
                    for j in range(NJ))

            init = tuple(buf[base, pl.ds(j * LANES, LANES)]
                         for j in range(NJ))
            accs = lax.fori_loop(1, S, row_body, init, unroll=8)
            for j in range(NJ):
                obuf[n, pl.ds(j * LANES, LANES)] = accs[j]
            return 0

        lax.fori_loop(0, CH, node_body, 0, unroll=False)

    for b in range(NB - 1):
        start_in(b, b)

    def outer(jj, _):
        for b in range(NB):
            j = jj * NB + b

            @pl.when(j + NB - 1 < SLOTS)
            def _():
                start_in(j + NB - 1, (b + NB - 1) % NB)

            wait_in(b)

            @pl.when(jj >= 1)
            def _():
                wait_out(b)

            compute(b)
            start_out(j, b)
        return 0

    lax.fori_loop(0, SLOTS // NB, outer, 0, unroll=False)
    for b in range(NB):
        wait_out(b)


def _sc_aggregate(src_neg, dst_neg):
    mesh = plsc.VectorSubcoreMesh(core_axis_name="c", subcore_axis_name="s")
    f = pl.kernel(
        _sc_agg_body,
        out_type=(jax.ShapeDtypeStruct((NSC, D), jnp.float32),
                  jax.ShapeDtypeStruct((NSC, D), jnp.float32)),
        mesh=mesh,
        scratch_types=(
            [pltpu.VMEM((CHUNK_ROWS, D), jnp.float32)] * NB
            + [pltpu.VMEM((CH, D), jnp.float32)] * NB
            + [pltpu.SemaphoreType.DMA] * (2 * NB)
        ),
    )
    return f(src_neg, dst_neg)


def _dot(x, w):
    return jax.lax.dot_general(x, w, (((1,), (0,)), ((), ())),
                               preferred_element_type=jnp.float32)


def _tc_fused_body(src_ref, sneg_ref, dst_ref, dneg_ref, w_ref,
                   osrc_ref, odst_ref):
    B = src_ref.shape[0]
    w = w_ref[...]
    sagg = jnp.mean(jnp.reshape(sneg_ref[...], (B, S, D)), axis=1)
    dagg = jnp.mean(jnp.reshape(dneg_ref[...], (B, S, D)), axis=1)
    xs = jnp.concatenate([src_ref[...], sagg], axis=1)
    xd = jnp.concatenate([dst_ref[...], dagg], axis=1)
    osrc_ref[...] = jnp.maximum(_dot(xs, w), 0.0)
    odst_ref[...] = jnp.maximum(_dot(xd, w), 0.0)


def _tc_fused(src, src_neg, dst, dst_neg, w):
    B = 400
    nb = NTC // B
    grid = (nb,)
    row_spec = pl.BlockSpec((B, D), lambda i: (NSC // B + i, 0))
    neg_spec = pl.BlockSpec((B * S, D), lambda i: (NSC // B + i, 0))
    w_spec = pl.BlockSpec((2 * D, DOUT), lambda i: (0, 0))
    out_spec = pl.BlockSpec((B, DOUT), lambda i: (NSC // B + i, 0))
    return pl.pallas_call(
        _tc_fused_body,
        grid=grid,
        in_specs=[row_spec, neg_spec, row_spec, neg_spec, w_spec],
        out_specs=[out_spec, out_spec],
        out_shape=(jax.ShapeDtypeStruct((N, DOUT), jnp.float32),
                   jax.ShapeDtypeStruct((N, DOUT), jnp.float32)),
    )(src, src_neg, dst, dst_neg, w)


def _tc_head_body(src_ref, ssum_ref, dst_ref, dsum_ref, w_ref,
                  _tail_src, _tail_dst, osrc_ref, odst_ref):
    w = w_ref[...]
    inv = jnp.float32(1.0 / S)
    xs = jnp.concatenate([src_ref[...], ssum_ref[...] * inv], axis=1)
    xd = jnp.concatenate([dst_ref[...], dsum_ref[...] * inv], axis=1)
    osrc_ref[...] = jnp.maximum(_dot(xs, w), 0.0)
    odst_ref[...] = jnp.maximum(_dot(xd, w), 0.0)


def _tc_head(src, src_sum, dst, dst_sum, w, tail_src, tail_dst):
    B = 400
    assert NSC % B == 0
    grid = (NSC // B,)
    row_spec = pl.BlockSpec((B, D), lambda i: (i, 0))
    w_spec = pl.BlockSpec((2 * D, DOUT), lambda i: (0, 0))
    any_spec = pl.BlockSpec(memory_space=pl.ANY)
    out_spec = pl.BlockSpec((B, DOUT), lambda i: (i, 0))
    return pl.pallas_call(
        _tc_head_body,
        grid=grid,
        in_specs=[row_spec, row_spec, row_spec, row_spec, w_spec,
                  any_spec, any_spec],
        out_specs=[out_spec, out_spec],
        out_shape=(jax.ShapeDtypeStruct((N, DOUT), jnp.float32),
                   jax.ShapeDtypeStruct((N, DOUT), jnp.float32)),
        input_output_aliases={5: 0, 6: 1},
    )(src, src_sum, dst, dst_sum, w, tail_src, tail_dst)


@jax.jit
def kernel(src, src_neg, dst, dst_neg, w):
    src_sum, dst_sum = _sc_aggregate(src_neg, dst_neg)
    tail_src, tail_dst = _tc_fused(src, src_neg, dst, dst_neg, w)
    out_src, out_dst = _tc_head(src, src_sum, dst, dst_sum, w,
                                tail_src, tail_dst)
    return (out_src, out_dst)


# speedup vs baseline: 1.2936x; 1.2936x over previous
import jax
import jax.numpy as jnp
from jax.experimental import pallas as pl

N = 10000
S = 32
D = 128
DOUT = 128
B = 400


def _dot(x, w):
    return jax.lax.dot_general(x, w, (((1,), (0,)), ((), ())),
                               preferred_element_type=jnp.float32)


def _fused_body(src_ref, sneg_ref, dst_ref, dneg_ref, w_ref,
                osrc_ref, odst_ref):
    w = w_ref[...]
    sagg = jnp.mean(jnp.reshape(sneg_ref[...], (B, S, D)), axis=1)
    dagg = jnp.mean(jnp.reshape(dneg_ref[...], (B, S, D)), axis=1)
    xs = jnp.concatenate([src_ref[...], sagg], axis=1)
    xd = jnp.concatenate([dst_ref[...], dagg], axis=1)
    osrc_ref[...] = jnp.maximum(_dot(xs, w), 0.0)
    odst_ref[...] = jnp.maximum(_dot(xd, w), 0.0)


@jax.jit
def kernel(src, src_neg, dst, dst_neg, w):
    grid = (N // B,)
    row_spec = pl.BlockSpec((B, D), lambda i: (i, 0))
    neg_spec = pl.BlockSpec((B * S, D), lambda i: (i, 0))
    w_spec = pl.BlockSpec((2 * D, DOUT), lambda i: (0, 0))
    out_spec = pl.BlockSpec((B, DOUT), lambda i: (i, 0))
    return pl.pallas_call(
        _fused_body,
        grid=grid,
        in_specs=[row_spec, neg_spec, row_spec, neg_spec, w_spec],
        out_specs=[out_spec, out_spec],
        out_shape=(jax.ShapeDtypeStruct((N, DOUT), jnp.float32),
                   jax.ShapeDtypeStruct((N, DOUT), jnp.float32)),
    )(src, src_neg, dst, dst_neg, w)


# speedup vs baseline: 1.3043x; 1.0083x over previous
import jax
import jax.numpy as jnp
from jax.experimental import pallas as pl

N = 10000
S = 32
D = 128
DOUT = 128
B = 200


def _dot(x, w):
    return jax.lax.dot_general(x, w, (((1,), (0,)), ((), ())),
                               preferred_element_type=jnp.float32)


def _fused_body(src_ref, sneg_ref, dst_ref, dneg_ref, w_ref,
                osrc_ref, odst_ref):
    w = w_ref[...]
    sagg = jnp.mean(jnp.reshape(sneg_ref[...], (B, S, D)), axis=1)
    dagg = jnp.mean(jnp.reshape(dneg_ref[...], (B, S, D)), axis=1)
    xs = jnp.concatenate([src_ref[...], sagg], axis=1)
    xd = jnp.concatenate([dst_ref[...], dagg], axis=1)
    osrc_ref[...] = jnp.maximum(_dot(xs, w), 0.0)
    odst_ref[...] = jnp.maximum(_dot(xd, w), 0.0)


@jax.jit
def kernel(src, src_neg, dst, dst_neg, w):
    grid = (N // B,)
    row_spec = pl.BlockSpec((B, D), lambda i: (i, 0))
    neg_spec = pl.BlockSpec((B * S, D), lambda i: (i, 0))
    w_spec = pl.BlockSpec((2 * D, DOUT), lambda i: (0, 0))
    out_spec = pl.BlockSpec((B, DOUT), lambda i: (i, 0))
    return pl.pallas_call(
        _fused_body,
        grid=grid,
        in_specs=[row_spec, neg_spec, row_spec, neg_spec, w_spec],
        out_specs=[out_spec, out_spec],
        out_shape=(jax.ShapeDtypeStruct((N, DOUT), jnp.float32),
                   jax.ShapeDtypeStruct((N, DOUT), jnp.float32)),
    )(src, src_neg, dst, dst_neg, w)
